# baseline (speedup 1.0000x reference)
import functools

import jax
import jax.numpy as jnp
from jax import lax
from jax.experimental import pallas as pl
from jax.experimental.pallas import tpu as pltpu
from jax.experimental.pallas import tpu_sc as plsc

BATCH = 16384
DIM = 32
LANES = 16
WIN = 128
PHASE = 8

_info = plsc.get_sparse_core_info()
NC = _info.num_cores
NS = _info.num_subcores
NW = NC * NS
B_PER_W = BATCH // NW
NGROUP = B_PER_W // LANES


def _bpr_body(u_hbm, i_hbm, j_hbm, ut_hbm, it_hbm, pos_hbm, neg_hbm,
              idx_u, idx_i, idx_j, wu, wi, wj, pos_v, neg_v, sem):
    wid = lax.axis_index("s") * NC + lax.axis_index("c")
    base = wid * B_PER_W

    pltpu.sync_copy(u_hbm.at[pl.ds(base, B_PER_W)], idx_u)
    pltpu.sync_copy(i_hbm.at[pl.ds(base, B_PER_W)], idx_i)
    pltpu.sync_copy(j_hbm.at[pl.ds(base, B_PER_W)], idx_j)

    lanes = lax.iota(jnp.int32, LANES)
    slot = lanes & (PHASE - 1)

    def fire_phase(vu, vi, vj, ph):
        for t in range(PHASE):
            k = ph * PHASE + t
            ou = pl.multiple_of((vu[k] >> 7) * WIN, WIN)
            oi = pl.multiple_of((vi[k] >> 7) * WIN, WIN)
            oj = pl.multiple_of((vj[k] >> 7) * WIN, WIN)
            pltpu.async_copy(ut_hbm.at[:, pl.ds(ou, WIN)], wu.at[t], sem)
            pltpu.async_copy(it_hbm.at[:, pl.ds(oi, WIN)], wi.at[t], sem)
            pltpu.async_copy(it_hbm.at[:, pl.ds(oj, WIN)], wj.at[t], sem)

    def drain_phase():
        src = ut_hbm.at[:, pl.ds(0, WIN)]
        for t in range(PHASE):
            pltpu.make_async_copy(src, wu.at[t], sem).wait()
            pltpu.make_async_copy(src, wi.at[t], sem).wait()
            pltpu.make_async_copy(src, wj.at[t], sem).wait()

    def extract_phase(rl_u, rl_i, rl_j, ph):
        perm = ph * PHASE + slot
        ru = rl_u.at[perm].get(mode="promise_in_bounds")
        ri = rl_i.at[perm].get(mode="promise_in_bounds")
        rj = rl_j.at[perm].get(mode="promise_in_bounds")
        accp = jnp.zeros((LANES,), jnp.float32)
        accn = jnp.zeros((LANES,), jnp.float32)
        for c in range(DIM):
            cvec = jnp.full((LANES,), c, jnp.int32)
            gu = plsc.load_gather(wu, [slot, cvec, ru])
            gi = plsc.load_gather(wi, [slot, cvec, ri])
            gj = plsc.load_gather(wj, [slot, cvec, rj])
            accp = accp + gu * gi
            accn = accn + gu * gj
        return accp, accn

    def group_body(g, carry):
        goff = g * LANES
        vu = idx_u[pl.ds(goff, LANES)]
        vi = idx_i[pl.ds(goff, LANES)]
        vj = idx_j[pl.ds(goff, LANES)]
        rl_u = vu & (WIN - 1)
        rl_i = vi & (WIN - 1)
        rl_j = vj & (WIN - 1)

        fire_phase(vu, vi, vj, 0)
        drain_phase()
        p0, n0 = extract_phase(rl_u, rl_i, rl_j, 0)
        fire_phase(vu, vi, vj, 1)
        drain_phase()
        p1, n1 = extract_phase(rl_u, rl_i, rl_j, 1)

        lo = lanes < PHASE
        pos_v[pl.ds(goff, LANES)] = jnp.where(lo, p0, p1)
        neg_v[pl.ds(goff, LANES)] = jnp.where(lo, n0, n1)
        return carry

    lax.fori_loop(0, NGROUP, group_body, 0)

    pltpu.sync_copy(pos_v, pos_hbm.at[pl.ds(base, B_PER_W)])
    pltpu.sync_copy(neg_v, neg_hbm.at[pl.ds(base, B_PER_W)])


@jax.jit
def _bpr_call(u, i, j, user_table, item_table):
    ut_t = user_table.T
    it_t = item_table.T
    mesh = plsc.VectorSubcoreMesh(core_axis_name="c", subcore_axis_name="s")
    f = functools.partial(
        pl.kernel,
        mesh=mesh,
        compiler_params=pltpu.CompilerParams(needs_layout_passes=False),
        out_type=[
            jax.ShapeDtypeStruct((BATCH,), jnp.float32),
            jax.ShapeDtypeStruct((BATCH,), jnp.float32),
        ],
        scratch_types=[
            pltpu.VMEM((B_PER_W,), jnp.int32),
            pltpu.VMEM((B_PER_W,), jnp.int32),
            pltpu.VMEM((B_PER_W,), jnp.int32),
            pltpu.VMEM((PHASE, DIM, WIN), jnp.float32),
            pltpu.VMEM((PHASE, DIM, WIN), jnp.float32),
            pltpu.VMEM((PHASE, DIM, WIN), jnp.float32),
            pltpu.VMEM((B_PER_W,), jnp.float32),
            pltpu.VMEM((B_PER_W,), jnp.float32),
            pltpu.SemaphoreType.DMA,
        ],
    )(_bpr_body)
    return f(u, i, j, ut_t, it_t)


def kernel(u, i, j, user_table, item_table):
    u = u.astype(jnp.int32)
    i = i.astype(jnp.int32)
    j = j.astype(jnp.int32)
    pos, neg = _bpr_call(u, i, j, user_table, item_table)
    return (pos, neg)
